# baseline (speedup 1.0000x reference)
import jax
import jax.numpy as jnp
from jax.experimental import pallas as pl
from jax.experimental.pallas import tpu as pltpu

_NUM_EXPERTS = 64
_SIZES = (512, 1536, 2048, 2048, 1536, 512)
_OFFS = tuple(sum(_SIZES[:k]) for k in range(len(_SIZES)))
_MAXS = max(_SIZES)
_NBUF = 3


def _top2(logits):
    e = logits.shape[-1]
    iota = jax.lax.broadcasted_iota(jnp.int32, logits.shape, 1)
    m1 = jnp.max(logits, axis=-1, keepdims=True)
    i1 = jnp.min(jnp.where(logits == m1, iota, e), axis=-1, keepdims=True)
    masked = jnp.where(iota == i1, -jnp.inf, logits)
    m2 = jnp.max(masked, axis=-1, keepdims=True)
    i2 = jnp.min(jnp.where(masked == m2, iota, e), axis=-1, keepdims=True)
    z = jnp.sum(jnp.exp(logits - m1), axis=-1, keepdims=True)
    idx = jnp.concatenate([i1, i2], axis=-1)
    score = jnp.concatenate([1.0 / z, jnp.exp(m2 - m1) / z], axis=-1)
    return idx, score


def _router_kernel(x_hbm, w_ref, b_ref, idx_ref, score_ref, b0, b1, b2c, sems):
    bufs = (b0, b1, b2c)

    def copy(k):
        slot = k % _NBUF
        sz = _SIZES[k]
        return pltpu.make_async_copy(
            x_hbm.at[pl.ds(_OFFS[k], sz), :],
            bufs[slot].at[pl.ds(0, sz), :],
            sems.at[slot],
        )

    copy(0).start()
    copy(1).start()
    for k in range(len(_SIZES)):
        copy(k).wait()
        if k + 2 < len(_SIZES):
            copy(k + 2).start()
        sz = _SIZES[k]
        logits = jax.lax.dot_general(
            bufs[k % _NBUF][0:sz, :], w_ref[...], (((1,), (1,)), ((), ())),
            preferred_element_type=jnp.float32,
        ) + b_ref[...]
        idx, score = _top2(logits)
        idx_ref[pl.ds(_OFFS[k], sz), :] = idx
        score_ref[pl.ds(_OFFS[k], sz), :] = score


@jax.jit
def kernel(x, W, b):
    tokens, d_model = x.shape
    b2 = b.reshape(1, _NUM_EXPERTS)
    idx, scores = pl.pallas_call(
        _router_kernel,
        in_specs=[
            pl.BlockSpec(memory_space=pltpu.HBM),
            pl.BlockSpec((_NUM_EXPERTS, d_model), lambda: (0, 0)),
            pl.BlockSpec((1, _NUM_EXPERTS), lambda: (0, 0)),
        ],
        out_specs=[
            pl.BlockSpec((tokens, 2), lambda: (0, 0)),
            pl.BlockSpec((tokens, 2), lambda: (0, 0)),
        ],
        out_shape=[
            jax.ShapeDtypeStruct((tokens, 2), jnp.int32),
            jax.ShapeDtypeStruct((tokens, 2), jnp.float32),
        ],
        scratch_shapes=[
            pltpu.VMEM((max(_SIZES[0], _SIZES[3]), d_model), jnp.float32),
            pltpu.VMEM((max(_SIZES[1], _SIZES[4]), d_model), jnp.float32),
            pltpu.VMEM((max(_SIZES[2], _SIZES[5]), d_model), jnp.float32),
            pltpu.SemaphoreType.DMA((_NBUF,)),
        ],
    )(x, W, b2)
    return idx, scores


# speedup vs baseline: 1.1133x; 1.1133x over previous
import jax
import jax.numpy as jnp
from jax.experimental import pallas as pl
from jax.experimental.pallas import tpu as pltpu

_NUM_EXPERTS = 64
_TM = 2048


def _top2(logits):
    e = logits.shape[-1]
    iota = jax.lax.broadcasted_iota(jnp.int32, logits.shape, 1)
    m1 = jnp.max(logits, axis=-1, keepdims=True)
    i1 = jnp.min(jnp.where(logits == m1, iota, e), axis=-1, keepdims=True)
    masked = jnp.where(iota == i1, -jnp.inf, logits)
    m2 = jnp.max(masked, axis=-1, keepdims=True)
    i2 = jnp.min(jnp.where(masked == m2, iota, e), axis=-1, keepdims=True)
    z = jnp.sum(jnp.exp(logits - m1), axis=-1, keepdims=True)
    idx = jnp.concatenate([i1, i2], axis=-1)
    score = jnp.concatenate([1.0 / z, jnp.exp(m2 - m1) / z], axis=-1)
    return idx, score


def _router_block(x_ref, w_ref, b_ref, idx_ref, score_ref, lg_ref):
    i = pl.program_id(0)
    nblk = pl.num_programs(0) - 1

    @pl.when(i > 0)
    def _epilogue():
        idx, score = _top2(lg_ref[...])
        idx_ref[...] = idx
        score_ref[...] = score

    @pl.when(i < nblk)
    def _matmul():
        lg_ref[...] = jax.lax.dot_general(
            x_ref[...], w_ref[...], (((1,), (1,)), ((), ())),
            preferred_element_type=jnp.float32,
        ) + b_ref[...]


@jax.jit
def kernel(x, W, b):
    tokens, d_model = x.shape
    nblk = tokens // _TM
    b2 = b.reshape(1, _NUM_EXPERTS)
    idx, scores = pl.pallas_call(
        _router_block,
        grid=(nblk + 1,),
        in_specs=[
            pl.BlockSpec((_TM, d_model), lambda i: (jnp.minimum(i, nblk - 1), 0)),
            pl.BlockSpec((_NUM_EXPERTS, d_model), lambda i: (0, 0)),
            pl.BlockSpec((1, _NUM_EXPERTS), lambda i: (0, 0)),
        ],
        out_specs=[
            pl.BlockSpec((_TM, 2), lambda i: (jnp.maximum(i - 1, 0), 0)),
            pl.BlockSpec((_TM, 2), lambda i: (jnp.maximum(i - 1, 0), 0)),
        ],
        out_shape=[
            jax.ShapeDtypeStruct((tokens, 2), jnp.int32),
            jax.ShapeDtypeStruct((tokens, 2), jnp.float32),
        ],
        scratch_shapes=[
            pltpu.VMEM((_TM, _NUM_EXPERTS), jnp.float32),
        ],
        compiler_params=pltpu.CompilerParams(
            dimension_semantics=("arbitrary",),
        ),
    )(x, W, b2)
    return idx, scores


# speedup vs baseline: 1.2234x; 1.0989x over previous
import jax
import jax.numpy as jnp
from jax.experimental import pallas as pl
from jax.experimental.pallas import tpu as pltpu

_NUM_EXPERTS = 64
_TM = 512
_NS = 4


def _top2(logits):
    e = logits.shape[-1]
    iota = jax.lax.broadcasted_iota(jnp.int32, logits.shape, 1)
    m1 = jnp.max(logits, axis=-1, keepdims=True)
    i1 = jnp.min(jnp.where(logits == m1, iota, e), axis=-1, keepdims=True)
    masked = jnp.where(iota == i1, -jnp.inf, logits)
    m2 = jnp.max(masked, axis=-1, keepdims=True)
    i2 = jnp.min(jnp.where(masked == m2, iota, e), axis=-1, keepdims=True)
    z = jnp.sum(jnp.exp(logits - m1), axis=-1, keepdims=True)
    idx = jnp.concatenate([i1, i2], axis=-1)
    score = jnp.concatenate([1.0 / z, jnp.exp(m2 - m1) / z], axis=-1)
    return idx, score


def _router_block(*refs):
    x_refs = refs[:_NS]
    w_ref, b_ref, idx_ref, score_ref = refs[_NS:]
    w = w_ref[...]
    bias = b_ref[...]
    for s in range(_NS):
        logits = jax.lax.dot_general(
            x_refs[s][...], w, (((1,), (1,)), ((), ())),
            preferred_element_type=jnp.float32,
        )
        logits = logits + bias
        idx, score = _top2(logits)
        idx_ref[pl.ds(s * _TM, _TM), :] = idx
        score_ref[pl.ds(s * _TM, _TM), :] = score


@jax.jit
def kernel(x, W, b):
    tokens, d_model = x.shape
    b2 = b.reshape(1, _NUM_EXPERTS)
    grid = (tokens // (_TM * _NS),)
    x_specs = [
        pl.BlockSpec((_TM, d_model), lambda i, s=s: (i * _NS + s, 0))
        for s in range(_NS)
    ]
    idx, scores = pl.pallas_call(
        _router_block,
        grid=grid,
        in_specs=x_specs + [
            pl.BlockSpec((_NUM_EXPERTS, d_model), lambda i: (0, 0)),
            pl.BlockSpec((1, _NUM_EXPERTS), lambda i: (0, 0)),
        ],
        out_specs=[
            pl.BlockSpec((_TM * _NS, 2), lambda i: (i, 0)),
            pl.BlockSpec((_TM * _NS, 2), lambda i: (i, 0)),
        ],
        out_shape=[
            jax.ShapeDtypeStruct((tokens, 2), jnp.int32),
            jax.ShapeDtypeStruct((tokens, 2), jnp.float32),
        ],
        compiler_params=pltpu.CompilerParams(
            dimension_semantics=("arbitrary",),
        ),
    )(*([x] * _NS), W, b2)
    return idx, scores
